# baseline (speedup 1.0000x reference)
import functools

import jax
import jax.numpy as jnp
from jax.experimental import pallas as pl
from jax.experimental.pallas import tpu as pltpu

_THRESHOLD = 1.5


def _nn_body(q_ref, k_ref, idx_ref, min_ref, *, n_valid, bk, nb):
    i = pl.program_id(1)
    q = q_ref[...]
    k = k_ref[...]
    m = jax.lax.dot_general(
        q, k, (((1,), (1,)), ((), ())),
        preferred_element_type=jnp.float32,
        precision=jax.lax.Precision.DEFAULT,
    )
    q_sq = jnp.sum(q * q, axis=1, keepdims=True)
    k_sq = jnp.sum(k * k, axis=1)[None, :]
    d = (q_sq + k_sq) - 2.0 * m

    lidx = jax.lax.broadcasted_iota(jnp.int32, d.shape, 1)
    limit = jnp.where(i == nb - 1, n_valid - i * bk, bk)
    d = jnp.where(lidx < limit, d, jnp.inf)

    bmin = jnp.min(d, axis=1, keepdims=True)
    cand = jnp.where(d == bmin, lidx, bk)
    barg = jnp.min(cand, axis=1, keepdims=True) + i * bk

    @pl.when(i == 0)
    def _init():
        min_ref[...] = bmin
        idx_ref[...] = barg

    @pl.when(i > 0)
    def _update():
        prev = min_ref[...]
        take = bmin < prev
        min_ref[...] = jnp.where(take, bmin, prev)
        idx_ref[...] = jnp.where(take, barg, idx_ref[...])

    @pl.when(i == nb - 1)
    def _final():
        idx_ref[...] = jnp.where(min_ref[...] > _THRESHOLD, -1, idx_ref[...])


def kernel(source_embs, embeddings):
    q, d_dim = source_embs.shape
    n_k, _ = embeddings.shape
    bk = 2048
    nb = (n_k + bk - 1) // bk
    pad = nb * bk - n_k
    if pad:
        embeddings = jnp.pad(embeddings, ((0, pad), (0, 0)))

    bq = q // 2
    body = functools.partial(_nn_body, n_valid=n_k, bk=bk, nb=nb)
    idx2, min2 = pl.pallas_call(
        body,
        grid=(2, nb),
        in_specs=[
            pl.BlockSpec((bq, d_dim), lambda j, i: (j, 0)),
            pl.BlockSpec((bk, d_dim), lambda j, i: (i, 0)),
        ],
        out_specs=[
            pl.BlockSpec((bq, 1), lambda j, i: (j, 0)),
            pl.BlockSpec((bq, 1), lambda j, i: (j, 0)),
        ],
        out_shape=[
            jax.ShapeDtypeStruct((q, 1), jnp.int32),
            jax.ShapeDtypeStruct((q, 1), jnp.float32),
        ],
        compiler_params=pltpu.CompilerParams(
            dimension_semantics=("parallel", "arbitrary"),
        ),
    )(source_embs, embeddings)
    return (idx2.reshape(q), min2.reshape(q))


# speedup vs baseline: 1.6178x; 1.6178x over previous
import functools

import jax
import jax.numpy as jnp
from jax.experimental import pallas as pl
from jax.experimental.pallas import tpu as pltpu

_THRESHOLD = 1.5


def _nn_body(q_ref, k_ref, idx_ref, min_ref, *, bk, nb):
    i = pl.program_id(0)
    q = q_ref[...]
    k = k_ref[...]
    m = jax.lax.dot_general(
        q, k, (((1,), (1,)), ((), ())),
        preferred_element_type=jnp.float32,
    )
    q_sq = jnp.sum(q * q, axis=1, keepdims=True)
    k_sq = jnp.sum(k * k, axis=1)[None, :]
    d = (q_sq + k_sq) - 2.0 * m

    bmin = jnp.min(d, axis=1, keepdims=True)
    lidx = jax.lax.broadcasted_iota(jnp.int32, d.shape, 1)
    cand = jnp.where(d == bmin, lidx, bk)
    barg = jnp.min(cand, axis=1, keepdims=True) + i * bk

    @pl.when(i == 0)
    def _init():
        min_ref[...] = bmin
        idx_ref[...] = barg

    @pl.when(i > 0)
    def _update():
        prev = min_ref[...]
        take = bmin < prev
        min_ref[...] = jnp.where(take, bmin, prev)
        idx_ref[...] = jnp.where(take, barg, idx_ref[...])

    @pl.when(i == nb - 1)
    def _final():
        idx_ref[...] = jnp.where(min_ref[...] > _THRESHOLD, -1, idx_ref[...])


def kernel(source_embs, embeddings):
    q, d_dim = source_embs.shape
    n_k, _ = embeddings.shape
    bk = 2000
    assert n_k % bk == 0
    nb = n_k // bk

    body = functools.partial(_nn_body, bk=bk, nb=nb)
    idx2, min2 = pl.pallas_call(
        body,
        grid=(nb,),
        in_specs=[
            pl.BlockSpec((q, d_dim), lambda i: (0, 0)),
            pl.BlockSpec((bk, d_dim), lambda i: (i, 0)),
        ],
        out_specs=[
            pl.BlockSpec((q, 1), lambda i: (0, 0)),
            pl.BlockSpec((q, 1), lambda i: (0, 0)),
        ],
        out_shape=[
            jax.ShapeDtypeStruct((q, 1), jnp.int32),
            jax.ShapeDtypeStruct((q, 1), jnp.float32),
        ],
        compiler_params=pltpu.CompilerParams(
            dimension_semantics=("arbitrary",),
        ),
    )(source_embs, embeddings)
    return (idx2.reshape(q), min2.reshape(q))


# speedup vs baseline: 1.7718x; 1.0952x over previous
import functools

import jax
import jax.numpy as jnp
from jax.experimental import pallas as pl
from jax.experimental.pallas import tpu as pltpu

_THRESHOLD = 1.5


def _nn_body(q_ref, k_ref, idx_ref, min_ref, sval, sidx, *, bk, nb):
    i = pl.program_id(0)
    q = q_ref[...]
    k = k_ref[...]
    nq = q.shape[0]
    m = jax.lax.dot_general(
        q, k, (((1,), (1,)), ((), ())),
        preferred_element_type=jnp.float32,
    )
    q_sq = jnp.sum(q * q, axis=1, keepdims=True)
    k_sq = jnp.sum(k * k, axis=1)[None, :]
    d = (q_sq + k_sq) - 2.0 * m

    bases = list(range(0, bk - 128, 128)) + [bk - 128]
    val = d[:, bases[0]:bases[0] + 128]
    base = jnp.zeros((nq, 128), jnp.float32)
    for b in bases[1:]:
        dj = d[:, b:b + 128]
        take = dj < val
        val = jnp.where(take, dj, val)
        base = jnp.where(take, jnp.float32(b), base)

    rm = jnp.min(val, axis=1, keepdims=True)
    lane = jax.lax.broadcasted_iota(jnp.int32, (nq, 128), 1).astype(jnp.float32)
    cand = jnp.where(val == rm, base + lane, jnp.float32(2 * bk))
    ri = jnp.min(cand, axis=1, keepdims=True) + jnp.float32(i * bk)

    @pl.when(i == 0)
    def _init():
        sval[...] = rm
        sidx[...] = ri

    @pl.when(i > 0)
    def _update():
        prev = sval[...]
        take = rm < prev
        sval[...] = jnp.where(take, rm, prev)
        sidx[...] = jnp.where(take, ri, sidx[...])

    @pl.when(i == nb - 1)
    def _final():
        mn = sval[...]
        ix = jnp.where(mn > _THRESHOLD, jnp.float32(-1), sidx[...])
        min_ref[...] = mn.reshape(nq)
        idx_ref[...] = ix.reshape(nq).astype(jnp.int32)


def kernel(source_embs, embeddings):
    q, d_dim = source_embs.shape
    n_k, _ = embeddings.shape
    bk = 2000
    assert n_k % bk == 0
    nb = n_k // bk

    body = functools.partial(_nn_body, bk=bk, nb=nb)
    idx1, min1 = pl.pallas_call(
        body,
        grid=(nb,),
        in_specs=[
            pl.BlockSpec((q, d_dim), lambda i: (0, 0)),
            pl.BlockSpec((bk, d_dim), lambda i: (i, 0)),
        ],
        out_specs=[
            pl.BlockSpec((q,), lambda i: (0,)),
            pl.BlockSpec((q,), lambda i: (0,)),
        ],
        out_shape=[
            jax.ShapeDtypeStruct((q,), jnp.int32),
            jax.ShapeDtypeStruct((q,), jnp.float32),
        ],
        scratch_shapes=[
            pltpu.VMEM((q, 1), jnp.float32),
            pltpu.VMEM((q, 1), jnp.float32),
        ],
        compiler_params=pltpu.CompilerParams(
            dimension_semantics=("arbitrary",),
        ),
    )(source_embs, embeddings)
    return (idx1, min1)


# speedup vs baseline: 1.7720x; 1.0001x over previous
import functools

import jax
import jax.numpy as jnp
from jax.experimental import pallas as pl
from jax.experimental.pallas import tpu as pltpu

_THRESHOLD = 1.5


def _nn_body(q_ref, k_ref, idx_ref, min_ref, sval, sidx, *, bk, nb):
    i = pl.program_id(0)
    q = q_ref[...]
    k = k_ref[...]
    nq = q.shape[0]
    m = jax.lax.dot_general(
        q, k, (((1,), (1,)), ((), ())),
        preferred_element_type=jnp.float32,
    )
    q_sq = jnp.sum(q * q, axis=1, keepdims=True)
    k_sq = jnp.sum(k * k, axis=1)[None, :]

    bases = list(range(0, bk - 128, 128)) + [bk - 128]

    def _dist(b):
        return (q_sq + k_sq[:, b:b + 128]) - 2.0 * m[:, b:b + 128]

    val = _dist(bases[0])
    base = jnp.zeros((nq, 128), jnp.float32)
    for b in bases[1:]:
        dj = _dist(b)
        take = dj < val
        val = jnp.where(take, dj, val)
        base = jnp.where(take, jnp.float32(b), base)

    rm = jnp.min(val, axis=1, keepdims=True)
    lane = jax.lax.broadcasted_iota(jnp.int32, (nq, 128), 1).astype(jnp.float32)
    cand = jnp.where(val == rm, base + lane, jnp.float32(2 * bk))
    ri = jnp.min(cand, axis=1, keepdims=True) + jnp.float32(i * bk)

    @pl.when(i == 0)
    def _init():
        sval[...] = rm
        sidx[...] = ri

    @pl.when(i > 0)
    def _update():
        prev = sval[...]
        take = rm < prev
        sval[...] = jnp.where(take, rm, prev)
        sidx[...] = jnp.where(take, ri, sidx[...])

    @pl.when(i == nb - 1)
    def _final():
        mn = sval[...]
        ix = jnp.where(mn > _THRESHOLD, jnp.float32(-1), sidx[...])
        min_ref[...] = mn.reshape(nq)
        idx_ref[...] = ix.reshape(nq).astype(jnp.int32)


def kernel(source_embs, embeddings):
    q, d_dim = source_embs.shape
    n_k, _ = embeddings.shape
    bk = 2000
    assert n_k % bk == 0
    nb = n_k // bk

    body = functools.partial(_nn_body, bk=bk, nb=nb)
    idx1, min1 = pl.pallas_call(
        body,
        grid=(nb,),
        in_specs=[
            pl.BlockSpec((q, d_dim), lambda i: (0, 0)),
            pl.BlockSpec((bk, d_dim), lambda i: (i, 0)),
        ],
        out_specs=[
            pl.BlockSpec((q,), lambda i: (0,)),
            pl.BlockSpec((q,), lambda i: (0,)),
        ],
        out_shape=[
            jax.ShapeDtypeStruct((q,), jnp.int32),
            jax.ShapeDtypeStruct((q,), jnp.float32),
        ],
        scratch_shapes=[
            pltpu.VMEM((q, 1), jnp.float32),
            pltpu.VMEM((q, 1), jnp.float32),
        ],
        compiler_params=pltpu.CompilerParams(
            dimension_semantics=("arbitrary",),
        ),
    )(source_embs, embeddings)
    return (idx1, min1)
